# baseline (speedup 1.0000x reference)
import functools

import jax
import jax.numpy as jnp
from jax.experimental import pallas as pl
from jax.experimental.pallas import tpu as pltpu

NB = 1024
CD = 256
BT = 2304
CHW = 256
NTOK = 16 * 576
NBLK = NTOK // BT


def _vq_kernel(x_ref, cbt_ref, cb_ref, out_ref, loss_ref, perp_ref,
               counts_ref, lsum_ref, c2_ref, cbtm2_ref):
    i = pl.program_id(0)
    x = x_ref[...]
    cbt = cbt_ref[...]

    @pl.when(i == 0)
    def _c2():
        c2_ref[...] = jnp.sum(cbt * cbt, axis=0, keepdims=True)
        cbtm2_ref[...] = cbt * -2.0

    x2 = jnp.sum(x * x, axis=1, keepdims=True)
    codes_c = jax.lax.broadcasted_iota(jnp.int32, (BT, CHW), 1)
    mn = None
    idx = None
    for k in range(NB // CHW):
        lo = k * CHW
        mm_c = jnp.dot(x, cbtm2_ref[:, lo:lo + CHW],
                       preferred_element_type=jnp.float32)
        dist_c = (x2 + mm_c) + c2_ref[:, lo:lo + CHW]
        mn_c = jnp.min(dist_c, axis=1, keepdims=True)
        idx_c = jnp.min(jnp.where(dist_c == mn_c, codes_c, NB),
                        axis=1, keepdims=True) + lo
        if k == 0:
            mn, idx = mn_c, idx_c
        else:
            idx = jnp.where(mn_c < mn, idx_c, idx)
            mn = jnp.minimum(mn_c, mn)

    codes = jax.lax.broadcasted_iota(jnp.int32, (BT, NB), 1)
    onehot = (codes == idx).astype(jnp.bfloat16)
    cb_bf = cb_ref[...].astype(jnp.bfloat16)

    x_d = jax.lax.dot_general(onehot, cb_bf, (((1,), (0,)), ((), ())),
                              preferred_element_type=jnp.float32)
    out_ref[...] = x_d

    blk_loss = jnp.sum(mn)
    blk_counts = jax.lax.dot_general(
        jnp.ones((1, BT), jnp.bfloat16), onehot, (((1,), (0,)), ((), ())),
        preferred_element_type=jnp.float32)

    @pl.when(i == 0)
    def _init():
        counts_ref[...] = blk_counts
        lsum_ref[0, 0] = blk_loss

    @pl.when(i > 0)
    def _acc():
        counts_ref[...] += blk_counts
        lsum_ref[0, 0] += blk_loss

    @pl.when(i == NBLK - 1)
    def _fin():
        counts = counts_ref[...]
        prob = counts / jnp.sum(counts)
        perp = jnp.exp(-jnp.sum(prob * jnp.log(prob + 1e-07)))
        perp_ref[...] = perp.reshape(1, 1)
        loss_ref[...] = (lsum_ref[0, 0] / jnp.float32(NTOK * CD)).reshape(1, 1)


@functools.partial(jax.jit, static_argnames=())
def kernel(x, codebook):
    N, T, C = x.shape
    xf = x.reshape(-1, C)
    cbt = codebook.T

    out, loss, perp = pl.pallas_call(
        _vq_kernel,
        grid=(NBLK,),
        in_specs=[
            pl.BlockSpec((BT, CD), lambda i: (i, 0)),
            pl.BlockSpec((CD, NB), lambda i: (0, 0)),
            pl.BlockSpec((NB, CD), lambda i: (0, 0)),
        ],
        out_specs=[
            pl.BlockSpec((BT, CD), lambda i: (i, 0)),
            pl.BlockSpec((1, 1), lambda i: (0, 0)),
            pl.BlockSpec((1, 1), lambda i: (0, 0)),
        ],
        out_shape=[
            jax.ShapeDtypeStruct((NTOK, CD), jnp.float32),
            jax.ShapeDtypeStruct((1, 1), jnp.float32),
            jax.ShapeDtypeStruct((1, 1), jnp.float32),
        ],
        scratch_shapes=[
            pltpu.VMEM((1, NB), jnp.float32),
            pltpu.SMEM((1, 1), jnp.float32),
            pltpu.VMEM((1, NB), jnp.float32),
            pltpu.VMEM((CD, NB), jnp.float32),
        ],
    )(xf, cbt, codebook)

    return (out.reshape(N, T, C), loss[0, 0], perp[0, 0])


# speedup vs baseline: 1.0917x; 1.0917x over previous
import functools

import jax
import jax.numpy as jnp
from jax.experimental import pallas as pl
from jax.experimental.pallas import tpu as pltpu

NB = 1024
CD = 256
BT = 2304
CHW = 512
NTOK = 16 * 576
NBLK = NTOK // BT


def _vq_kernel(x_ref, cbt_ref, cb_ref, out_ref, loss_ref, perp_ref,
               counts_ref, lsum_ref, c2_ref, cbtm2_ref):
    i = pl.program_id(0)
    x = x_ref[...]
    cbt = cbt_ref[...]

    @pl.when(i == 0)
    def _c2():
        c2_ref[...] = jnp.sum(cbt * cbt, axis=0, keepdims=True)
        cbtm2_ref[...] = cbt * -2.0

    x2 = jnp.sum(x * x, axis=1, keepdims=True)
    codes_c = jax.lax.broadcasted_iota(jnp.int32, (BT, CHW), 1)
    mn = None
    idx = None
    for k in range(NB // CHW):
        lo = k * CHW
        mm_c = jnp.dot(x, cbtm2_ref[:, lo:lo + CHW],
                       preferred_element_type=jnp.float32)
        dist_c = (x2 + mm_c) + c2_ref[:, lo:lo + CHW]
        mn_c = jnp.min(dist_c, axis=1, keepdims=True)
        idx_c = jnp.min(jnp.where(dist_c == mn_c, codes_c, NB),
                        axis=1, keepdims=True) + lo
        if k == 0:
            mn, idx = mn_c, idx_c
        else:
            idx = jnp.where(mn_c < mn, idx_c, idx)
            mn = jnp.minimum(mn_c, mn)

    codes = jax.lax.broadcasted_iota(jnp.int32, (BT, NB), 1)
    onehot = (codes == idx).astype(jnp.bfloat16)
    cb_bf = cb_ref[...].astype(jnp.bfloat16)

    x_d = jax.lax.dot_general(onehot, cb_bf, (((1,), (0,)), ((), ())),
                              preferred_element_type=jnp.float32)
    out_ref[...] = x_d

    blk_loss = jnp.sum(mn)
    blk_counts = jax.lax.dot_general(
        jnp.ones((1, BT), jnp.bfloat16), onehot, (((1,), (0,)), ((), ())),
        preferred_element_type=jnp.float32)

    @pl.when(i == 0)
    def _init():
        counts_ref[...] = blk_counts
        lsum_ref[0, 0] = blk_loss

    @pl.when(i > 0)
    def _acc():
        counts_ref[...] += blk_counts
        lsum_ref[0, 0] += blk_loss

    @pl.when(i == NBLK - 1)
    def _fin():
        counts = counts_ref[...]
        prob = counts / jnp.sum(counts)
        perp = jnp.exp(-jnp.sum(prob * jnp.log(prob + 1e-07)))
        perp_ref[...] = perp.reshape(1, 1)
        loss_ref[...] = (lsum_ref[0, 0] / jnp.float32(NTOK * CD)).reshape(1, 1)


@functools.partial(jax.jit, static_argnames=())
def kernel(x, codebook):
    N, T, C = x.shape
    xf = x.reshape(-1, C)
    cbt = codebook.T

    out, loss, perp = pl.pallas_call(
        _vq_kernel,
        grid=(NBLK,),
        in_specs=[
            pl.BlockSpec((BT, CD), lambda i: (i, 0)),
            pl.BlockSpec((CD, NB), lambda i: (0, 0)),
            pl.BlockSpec((NB, CD), lambda i: (0, 0)),
        ],
        out_specs=[
            pl.BlockSpec((BT, CD), lambda i: (i, 0)),
            pl.BlockSpec((1, 1), lambda i: (0, 0)),
            pl.BlockSpec((1, 1), lambda i: (0, 0)),
        ],
        out_shape=[
            jax.ShapeDtypeStruct((NTOK, CD), jnp.float32),
            jax.ShapeDtypeStruct((1, 1), jnp.float32),
            jax.ShapeDtypeStruct((1, 1), jnp.float32),
        ],
        scratch_shapes=[
            pltpu.VMEM((1, NB), jnp.float32),
            pltpu.SMEM((1, 1), jnp.float32),
            pltpu.VMEM((1, NB), jnp.float32),
            pltpu.VMEM((CD, NB), jnp.float32),
        ],
    )(xf, cbt, codebook)

    return (out.reshape(N, T, C), loss[0, 0], perp[0, 0])


# speedup vs baseline: 1.2431x; 1.1387x over previous
import functools

import jax
import jax.numpy as jnp
from jax.experimental import pallas as pl
from jax.experimental.pallas import tpu as pltpu

NB = 1024
CD = 256
BT = 2304
HB = BT // 2
NTOK = 16 * 576
NBLK = NTOK // BT


def _vq_kernel(x_ref, cbt_ref, cb_ref, out_ref, loss_ref, perp_ref,
               counts_ref, lsum_ref, c2_ref, cbtm2_ref):
    i = pl.program_id(0)
    cbt = cbt_ref[...]

    @pl.when(i == 0)
    def _c2():
        c2_ref[...] = jnp.sum(cbt * cbt, axis=0, keepdims=True)
        cbtm2_ref[...] = cbt * -2.0

    cb_bf = cb_ref[...].astype(jnp.bfloat16)
    codes = jax.lax.broadcasted_iota(jnp.int32, (HB, NB), 1)

    blk_loss = jnp.float32(0.0)
    blk_counts = jnp.zeros((1, NB), jnp.float32)
    for h in range(BT // HB):
        x = x_ref[h * HB:(h + 1) * HB, :]

        mm2 = jnp.dot(x, cbtm2_ref[...], preferred_element_type=jnp.float32)
        x2 = jnp.sum(x * x, axis=1, keepdims=True)
        dist = (x2 + mm2) + c2_ref[...]

        mn = jnp.min(dist, axis=1, keepdims=True)
        idx = jnp.min(jnp.where(dist == mn, codes, NB), axis=1, keepdims=True)
        onehot = (codes == idx).astype(jnp.bfloat16)

        x_d = jax.lax.dot_general(onehot, cb_bf, (((1,), (0,)), ((), ())),
                                  preferred_element_type=jnp.float32)
        out_ref[h * HB:(h + 1) * HB, :] = x_d

        blk_loss = blk_loss + jnp.sum(mn)
        blk_counts = blk_counts + jax.lax.dot_general(
            jnp.ones((1, HB), jnp.bfloat16), onehot, (((1,), (0,)), ((), ())),
            preferred_element_type=jnp.float32)

    @pl.when(i == 0)
    def _init():
        counts_ref[...] = blk_counts
        lsum_ref[0, 0] = blk_loss

    @pl.when(i > 0)
    def _acc():
        counts_ref[...] += blk_counts
        lsum_ref[0, 0] += blk_loss

    @pl.when(i == NBLK - 1)
    def _fin():
        counts = counts_ref[...]
        prob = counts / jnp.sum(counts)
        perp = jnp.exp(-jnp.sum(prob * jnp.log(prob + 1e-07)))
        perp_ref[...] = perp.reshape(1, 1)
        loss_ref[...] = (lsum_ref[0, 0] / jnp.float32(NTOK * CD)).reshape(1, 1)


@functools.partial(jax.jit, static_argnames=())
def kernel(x, codebook):
    N, T, C = x.shape
    xf = x.reshape(-1, C)
    cbt = codebook.T

    out, loss, perp = pl.pallas_call(
        _vq_kernel,
        grid=(NBLK,),
        in_specs=[
            pl.BlockSpec((BT, CD), lambda i: (i, 0)),
            pl.BlockSpec((CD, NB), lambda i: (0, 0)),
            pl.BlockSpec((NB, CD), lambda i: (0, 0)),
        ],
        out_specs=[
            pl.BlockSpec((BT, CD), lambda i: (i, 0)),
            pl.BlockSpec((1, 1), lambda i: (0, 0)),
            pl.BlockSpec((1, 1), lambda i: (0, 0)),
        ],
        out_shape=[
            jax.ShapeDtypeStruct((NTOK, CD), jnp.float32),
            jax.ShapeDtypeStruct((1, 1), jnp.float32),
            jax.ShapeDtypeStruct((1, 1), jnp.float32),
        ],
        scratch_shapes=[
            pltpu.VMEM((1, NB), jnp.float32),
            pltpu.SMEM((1, 1), jnp.float32),
            pltpu.VMEM((1, NB), jnp.float32),
            pltpu.VMEM((CD, NB), jnp.float32),
        ],
    )(xf, cbt, codebook)

    return (out.reshape(N, T, C), loss[0, 0], perp[0, 0])
